# Initial kernel scaffold; baseline (speedup 1.0000x reference)
import jax
import jax.numpy as jnp
from jax.experimental import pallas as pl


def kernel(t, x, edge_index, W_sheaf, W_left, W_right):
    raise NotImplementedError("write your pallas kernel here")


# speedup vs baseline: 9.1972x; 9.1972x over previous
import jax
import jax.numpy as jnp
from jax import lax
from jax.experimental import pallas as pl
from jax.experimental.pallas import tpu as pltpu
from jax.experimental.pallas import tpu_sc as plsc

N = 10000
D = 2
H = 64
E = 320000

NC = 2
NS = 16
L = 16
NW = NC * NS

C = 2000
NCH = E // C
NSUB = E // (NW * C)
CPT = (D * H) // NW

_i32 = jnp.int32
_f32 = jnp.float32


def _mesh():
    return plsc.VectorSubcoreMesh(
        core_axis_name="c", subcore_axis_name="s", num_cores=NC, num_subcores=NS
    )


def _tanh(z):
    e = jnp.exp(2.0 * z)
    return 1.0 - 2.0 / (e + 1.0)


def _dinv_rows(dinv):
    return jnp.concatenate(
        [jnp.broadcast_to(dinv[0:1, :], (H, N)),
         jnp.broadcast_to(dinv[1:2, :], (H, N))], axis=0)


def _dense_body(xt_ref, wpqt_ref, wbigt_ref, pqt_ref, nxwt_ref):
    xt = xt_ref[...]
    pqt_ref[...] = jnp.dot(wpqt_ref[...], xt, preferred_element_type=_f32)
    nxwt_ref[...] = -jnp.dot(wbigt_ref[...], xt, preferred_element_type=_f32)


def _pass1_body(pq_hbm, src_hbm, dst_hbm, zeros_hbm,
                deg_hbm, prod_hbm,
                pq_v, src_v, dst_v, prv, deg_v):
    c = lax.axis_index("c")
    s = lax.axis_index("s")
    wid = c * NS + s

    pltpu.sync_copy(pq_hbm, pq_v)
    pltpu.sync_copy(zeros_hbm.at[pl.ds(0, 2 * N)], deg_v)

    @pl.loop(0, NSUB)
    def _sub(j):
        ch = wid * NSUB + j
        pltpu.sync_copy(src_hbm.at[ch], src_v)
        pltpu.sync_copy(dst_hbm.at[ch], dst_v)

        @pl.loop(0, C // L)
        def _grp(i):
            off = i * L
            sv = src_v[pl.ds(off, L)]
            dv = dst_v[pl.ds(off, L)]
            p0 = plsc.load_gather(pq_v, [sv])
            p1 = plsc.load_gather(pq_v, [sv + N])
            p2 = plsc.load_gather(pq_v, [sv + 2 * N])
            p3 = plsc.load_gather(pq_v, [sv + 3 * N])
            q0 = plsc.load_gather(pq_v, [dv + 4 * N])
            q1 = plsc.load_gather(pq_v, [dv + 5 * N])
            q2 = plsc.load_gather(pq_v, [dv + 6 * N])
            q3 = plsc.load_gather(pq_v, [dv + 7 * N])
            fs0 = _tanh(p0 + q0)
            fs1 = _tanh(p1 + q1)
            fd0 = _tanh(p2 + q2)
            fd1 = _tanh(p3 + q3)
            prv[pl.ds(off, L)] = -(fs0 * fd0)
            prv[pl.ds(C + off, L)] = -(fs1 * fd1)
            plsc.addupdate_scatter(deg_v, [sv], fs0 * fs0)
            plsc.addupdate_scatter(deg_v, [sv + N], fs1 * fs1)
            plsc.addupdate_scatter(deg_v, [dv], fd0 * fd0)
            plsc.addupdate_scatter(deg_v, [dv + N], fd1 * fd1)

        pltpu.sync_copy(prv, prod_hbm.at[ch])

    pltpu.sync_copy(deg_v, deg_hbm.at[wid])


def _dinv_body(degp_ref, nxwt_ref, dinv_ref, nxwdt_ref):
    dsum = jnp.sum(degp_ref[...], axis=0) + 1e-6
    dinv = lax.rsqrt(dsum)
    dinv_ref[...] = dinv
    nxwdt_ref[...] = nxwt_ref[...] * _dinv_rows(dinv)


def _pass2_body(tbl_hbm, src_hbm, dst_hbm, prod_hbm, zeros_hbm,
                acct_hbm,
                tbl_v, acc_v, src_v, dst_v, np_v):
    c = lax.axis_index("c")
    s = lax.axis_index("s")
    wid = c * NS + s

    pltpu.sync_copy(tbl_hbm.at[wid], tbl_v)
    pltpu.sync_copy(zeros_hbm, acc_v)

    @pl.loop(0, NCH)
    def _chunk(ch):
        pltpu.sync_copy(src_hbm.at[ch], src_v)
        pltpu.sync_copy(dst_hbm.at[ch], dst_v)
        pltpu.sync_copy(prod_hbm.at[ch * 2 + c], np_v)

        @pl.loop(0, C // L)
        def _grp(i):
            off = i * L
            sv = src_v[pl.ds(off, L)]
            dv = dst_v[pl.ds(off, L)]
            npv = np_v[pl.ds(off, L)]
            for cc in range(CPT):
                gv = plsc.load_gather(tbl_v, [dv + cc * N])
                plsc.addupdate_scatter(acc_v, [sv + cc * N], gv * npv)

    pltpu.sync_copy(acc_v, acct_hbm.at[wid])


def _final_body(acc_ref, nxwt_ref, dinv_ref, yt_ref):
    yt_ref[...] = nxwt_ref[...] + acc_ref[...] * _dinv_rows(dinv_ref[...])


def kernel(t, x, edge_index, W_sheaf, W_left, W_right):
    x128t = x.reshape(N, D * H).T
    src = edge_index[0].astype(_i32).reshape(NCH, C)
    dst = edge_index[1].astype(_i32).reshape(NCH, C)

    wpqt = jnp.concatenate(
        [W_sheaf[: D * H, :].T, W_sheaf[D * H :, :].T], axis=0)
    wbigt = jnp.kron(W_left, W_right)
    zeros4n = jnp.zeros((CPT * N,), dtype=_f32)

    pqt, nxwt = pl.pallas_call(
        _dense_body,
        out_shape=[
            jax.ShapeDtypeStruct((2 * D * D, N), _f32),
            jax.ShapeDtypeStruct((D * H, N), _f32),
        ],
    )(x128t, wpqt, wbigt)

    pass1 = pl.kernel(
        _pass1_body,
        out_type=[
            jax.ShapeDtypeStruct((NW, 2 * N), _f32),
            jax.ShapeDtypeStruct((NCH, 2 * C), _f32),
        ],
        mesh=_mesh(),
        scratch_types=[
            pltpu.VMEM((2 * D * D * N,), _f32),
            pltpu.VMEM((C,), _i32),
            pltpu.VMEM((C,), _i32),
            pltpu.VMEM((2 * C,), _f32),
            pltpu.VMEM((2 * N,), _f32),
        ],
        compiler_params=pltpu.CompilerParams(needs_layout_passes=False),
    )
    degp, prod = pass1(pqt.reshape(-1), src, dst, zeros4n)

    dinv, nxwdt = pl.pallas_call(
        _dinv_body,
        out_shape=[
            jax.ShapeDtypeStruct((D, N), _f32),
            jax.ShapeDtypeStruct((D * H, N), _f32),
        ],
    )(degp.reshape(NW, D, N), nxwt)

    pass2 = pl.kernel(
        _pass2_body,
        out_type=jax.ShapeDtypeStruct((NW, CPT * N), _f32),
        mesh=_mesh(),
        scratch_types=[
            pltpu.VMEM((CPT * N,), _f32),
            pltpu.VMEM((CPT * N,), _f32),
            pltpu.VMEM((C,), _i32),
            pltpu.VMEM((C,), _i32),
            pltpu.VMEM((C,), _f32),
        ],
        compiler_params=pltpu.CompilerParams(needs_layout_passes=False),
    )
    acct = pass2(nxwdt.reshape(NW, CPT * N), src, dst,
                 prod.reshape(NCH * 2, C), zeros4n)

    yt = pl.pallas_call(
        _final_body,
        out_shape=jax.ShapeDtypeStruct((D * H, N), _f32),
    )(acct.reshape(D * H, N), nxwt, dinv)
    return yt.T.reshape(N * D, H)
